# baseline (speedup 1.0000x reference)
import functools

import jax
import jax.numpy as jnp
from jax import lax
from jax.experimental import pallas as pl
from jax.experimental.pallas import tpu as pltpu
from jax.experimental.pallas import tpu_sc as plsc

TILE_T = 512
TILE_M = 512
TILE_F = 512


def _router_body(x_ref, wr_ref, br_ref, gpair_ref, pos1_ref, pos2_ref,
                 gids_ref, cnt_s, base_s, *, n_e, tile_m, n_tiles):
    s = pl.program_id(0)
    t = pl.program_id(1)
    tile_t = x_ref.shape[0]

    @pl.when((s == 0) & (t == 0))
    def _():
        cnt_s[...] = jnp.zeros_like(cnt_s)

    @pl.when((s == 1) & (t == 0))
    def _():
        base_s[...] = jnp.zeros_like(base_s)

    logits = jnp.dot(x_ref[...], wr_ref[...],
                     preferred_element_type=jnp.float32) + br_ref[...]
    col = lax.broadcasted_iota(jnp.int32, logits.shape, 1)
    m1 = jnp.max(logits, axis=1, keepdims=True)
    e1 = jnp.min(jnp.where(logits == m1, col, n_e), axis=1, keepdims=True)
    dropped = jnp.where(col == e1, -1e30, logits)
    m2 = jnp.max(dropped, axis=1, keepdims=True)
    e2 = jnp.min(jnp.where(dropped == m2, col, n_e), axis=1, keepdims=True)

    g1 = 1.0 / (1.0 + jnp.exp(m2 - m1))
    gpair_ref[...] = jnp.concatenate([g1, 1.0 - g1], axis=1)

    sel1 = (col == e1).astype(jnp.float32)
    sel2 = (col == e2).astype(jnp.float32)
    o12 = sel1 + sel2
    tile_cnt = jnp.sum(o12, axis=0, keepdims=True)

    @pl.when(s == 0)
    def _():
        cnt_s[...] += tile_cnt
        pos1_ref[...] = jnp.zeros_like(pos1_ref)
        pos2_ref[...] = jnp.zeros_like(pos2_ref)
        gids_ref[...] = jnp.zeros_like(gids_ref)

    @pl.when(s == 1)
    def _():
        cnt = cnt_s[...]
        cnt_pad = jnp.ceil(cnt * (1.0 / tile_m)) * tile_m
        r = lax.broadcasted_iota(jnp.int32, (n_e, n_e), 0)
        c = lax.broadcasted_iota(jnp.int32, (n_e, n_e), 1)
        tri = (r < c).astype(jnp.float32)
        off = jnp.dot(cnt_pad, tri, preferred_element_type=jnp.float32)
        ends = off + cnt_pad

        tcol = lax.broadcasted_iota(jnp.int32, (gids_ref.shape[0], 1), 0)
        starts = (tcol * tile_m).astype(jnp.float32)
        gid = jnp.sum((ends <= starts).astype(jnp.float32),
                      axis=1, keepdims=True)
        gids_ref[...] = jnp.minimum(gid, n_e - 1).astype(jnp.int32)

        rr = lax.broadcasted_iota(jnp.int32, (tile_t, tile_t), 0)
        cc = lax.broadcasted_iota(jnp.int32, (tile_t, tile_t), 1)
        lstrict = (cc < rr).astype(jnp.float32)
        cum = jnp.dot(lstrict, o12, preferred_element_type=jnp.float32)
        dst = off + base_s[...] + cum
        val1 = jnp.sum(jnp.where(col == e1, dst, 0.0), axis=1, keepdims=True)
        val2 = jnp.sum(jnp.where(col == e2, dst, 0.0), axis=1, keepdims=True)
        pos1_ref[...] = val1.astype(jnp.int32)[None, :, 0:1]
        pos2_ref[...] = val2.astype(jnp.int32)[None, :, 0:1]
        base_s[...] += tile_cnt


def _router(x2d, Wr, br, n_tiles_m):
    T, D = x2d.shape
    E = Wr.shape[1]
    nt = T // TILE_T
    ntm_pad = max(32, n_tiles_m)
    gpair, pos1, pos2, gids = pl.pallas_call(
        functools.partial(_router_body, n_e=E, tile_m=TILE_M,
                          n_tiles=n_tiles_m),
        grid=(2, nt),
        in_specs=[
            pl.BlockSpec((TILE_T, D), lambda s, t: (t, 0)),
            pl.BlockSpec((D, E), lambda s, t: (0, 0)),
            pl.BlockSpec((1, E), lambda s, t: (0, 0)),
        ],
        out_specs=[
            pl.BlockSpec((TILE_T, 2), lambda s, t: (t, 0)),
            pl.BlockSpec((1, TILE_T, 1), lambda s, t: (t, 0, 0)),
            pl.BlockSpec((1, TILE_T, 1), lambda s, t: (t, 0, 0)),
            pl.BlockSpec((ntm_pad, 1), lambda s, t: (0, 0)),
        ],
        out_shape=[
            jax.ShapeDtypeStruct((T, 2), jnp.float32),
            jax.ShapeDtypeStruct((nt, TILE_T, 1), jnp.int32),
            jax.ShapeDtypeStruct((nt, TILE_T, 1), jnp.int32),
            jax.ShapeDtypeStruct((ntm_pad, 1), jnp.int32),
        ],
        scratch_shapes=[
            pltpu.VMEM((1, E), jnp.float32),
            pltpu.VMEM((1, E), jnp.float32),
        ],
        compiler_params=pltpu.CompilerParams(
            dimension_semantics=("arbitrary", "arbitrary"),
        ),
    )(x2d, Wr, br.reshape(1, E))
    return (gpair, pos1.reshape(T), pos2.reshape(T),
            gids.reshape(ntm_pad)[:n_tiles_m])


def _make_dispatch(T, D, xs_rows):
    info = plsc.get_sparse_core_info()
    nw = info.num_cores * info.num_subcores
    tok_w = T // nw
    chunk = 64
    n_chunks = tok_w // chunk
    assert tok_w % chunk == 0
    mesh = plsc.VectorSubcoreMesh(core_axis_name="c", subcore_axis_name="s")

    @functools.partial(
        pl.kernel, mesh=mesh,
        out_type=jax.ShapeDtypeStruct((xs_rows, D), jnp.float32),
        scratch_types=[
            pltpu.VMEM((n_chunks, chunk), jnp.int32),
            pltpu.VMEM((n_chunks, chunk), jnp.int32),
            pltpu.VMEM((chunk, D), jnp.float32),
            pltpu.SemaphoreType.DMA,
        ],
        compiler_params=pltpu.CompilerParams(needs_layout_passes=False),
    )
    def dispatch(x_hbm, pos1_hbm, pos2_hbm, xs_hbm,
                 idx1_v, idx2_v, rows_v, sem):
        wid = lax.axis_index("s") * info.num_cores + lax.axis_index("c")
        pltpu.sync_copy(pos1_hbm.at[pl.ds(wid * n_chunks, n_chunks)], idx1_v)
        pltpu.sync_copy(pos2_hbm.at[pl.ds(wid * n_chunks, n_chunks)], idx2_v)
        for c in range(n_chunks):
            pltpu.sync_copy(
                x_hbm.at[pl.ds(wid * tok_w + c * chunk, chunk)], rows_v)
            cp1 = pltpu.async_copy(rows_v, xs_hbm.at[idx1_v.at[c]], sem)
            cp2 = pltpu.async_copy(rows_v, xs_hbm.at[idx2_v.at[c]], sem)
            cp1.wait()
            cp2.wait()

    return dispatch


def _mlp_body(gids_ref, xs_ref, w1_ref, b1_ref, w2_ref, b2_ref, out_ref):
    f = pl.program_id(1)

    @pl.when(f == 0)
    def _():
        out_ref[...] = jnp.broadcast_to(b2_ref[0], out_ref.shape)

    x16 = xs_ref[...].astype(jnp.bfloat16)
    h = jnp.maximum(
        jnp.dot(x16, w1_ref[0], preferred_element_type=jnp.float32)
        + b1_ref[0], 0.0)
    out_ref[...] += jnp.dot(h.astype(jnp.bfloat16), w2_ref[0],
                            preferred_element_type=jnp.float32)


def _grouped_mlp(Xs, W1, b1, W2, b2, gids, n_tiles_m):
    E, D, F = W1.shape
    nf = F // TILE_F
    grid_spec = pltpu.PrefetchScalarGridSpec(
        num_scalar_prefetch=1,
        grid=(n_tiles_m, nf),
        in_specs=[
            pl.BlockSpec((TILE_M, D), lambda t, f, g: (t, 0)),
            pl.BlockSpec((1, D, TILE_F), lambda t, f, g: (g[t], 0, f)),
            pl.BlockSpec((1, 1, TILE_F), lambda t, f, g: (g[t], 0, f)),
            pl.BlockSpec((1, TILE_F, D), lambda t, f, g: (g[t], f, 0)),
            pl.BlockSpec((1, 1, D), lambda t, f, g: (g[t], 0, 0)),
        ],
        out_specs=pl.BlockSpec((TILE_M, D), lambda t, f, g: (t, 0)),
    )
    return pl.pallas_call(
        _mlp_body,
        grid_spec=grid_spec,
        out_shape=jax.ShapeDtypeStruct((n_tiles_m * TILE_M, D), jnp.float32),
        compiler_params=pltpu.CompilerParams(
            dimension_semantics=("arbitrary", "arbitrary"),
        ),
    )(gids, Xs, W1.astype(jnp.bfloat16), b1.reshape(E, 1, F),
      W2.astype(jnp.bfloat16), b2.reshape(E, 1, D))


def _make_combine_gather(T, D, ys_rows):
    info = plsc.get_sparse_core_info()
    nw = info.num_cores * info.num_subcores
    tok_w = T // nw
    chunk = 64
    n_chunks = tok_w // chunk
    assert tok_w % chunk == 0
    mesh = plsc.VectorSubcoreMesh(core_axis_name="c", subcore_axis_name="s")

    @functools.partial(
        pl.kernel, mesh=mesh,
        out_type=[jax.ShapeDtypeStruct((T, D), jnp.float32),
                  jax.ShapeDtypeStruct((T, D), jnp.float32)],
        scratch_types=[
            pltpu.VMEM((tok_w,), jnp.int32),
            pltpu.VMEM((chunk, D), jnp.float32),
            pltpu.SemaphoreType.DMA,
        ],
        compiler_params=pltpu.CompilerParams(needs_layout_passes=False),
    )
    def combine(ys_hbm, pos1_hbm, pos2_hbm, y1_hbm, y2_hbm,
                idx_v, rows_v, sem):
        wid = lax.axis_index("s") * info.num_cores + lax.axis_index("c")
        base = wid * tok_w
        for pos_hbm, y_hbm in ((pos1_hbm, y1_hbm), (pos2_hbm, y2_hbm)):
            pltpu.sync_copy(pos_hbm.at[pl.ds(base, tok_w)], idx_v)
            for c in range(n_chunks):
                idx = idx_v.at[pl.ds(c * chunk, chunk)]
                pltpu.async_copy(ys_hbm.at[idx], rows_v, sem).wait()
                pltpu.sync_copy(rows_v,
                                y_hbm.at[pl.ds(base + c * chunk, chunk)])

    return combine


def _combine_body(g_ref, y1_ref, y2_ref, out_ref):
    g = g_ref[...]
    out_ref[...] = g[:, 0:1] * y1_ref[...] + g[:, 1:2] * y2_ref[...]


def _combine(gpair, Y1, Y2):
    T, D = Y1.shape
    nt = T // TILE_T
    return pl.pallas_call(
        _combine_body,
        grid=(nt,),
        in_specs=[
            pl.BlockSpec((TILE_T, 2), lambda t: (t, 0)),
            pl.BlockSpec((TILE_T, D), lambda t: (t, 0)),
            pl.BlockSpec((TILE_T, D), lambda t: (t, 0)),
        ],
        out_specs=pl.BlockSpec((TILE_T, D), lambda t: (t, 0)),
        out_shape=jax.ShapeDtypeStruct((T, D), jnp.float32),
    )(gpair, Y1, Y2)


def kernel(x, Wr, br, W1, b1, W2, b2):
    B, S, D = x.shape
    E = Wr.shape[1]
    T = B * S
    x2d = x.reshape(T, D)

    n_tiles_m = 2 * T // TILE_M + E - 1
    xs_rows = n_tiles_m * TILE_M

    gpair, pos1, pos2, gids = _router(x2d, Wr, br, n_tiles_m)
    Xs = _make_dispatch(T, D, xs_rows)(
        x2d, pos1.reshape(T // 64, 64), pos2.reshape(T // 64, 64))
    Ys = _grouped_mlp(Xs, W1, b1, W2, b2, gids, n_tiles_m)
    Y1, Y2 = _make_combine_gather(T, D, Ys.shape[0])(Ys, pos1, pos2)
    out = _combine(gpair, Y1, Y2)
    return out.reshape(B, S, D)


# speedup vs baseline: 1.1166x; 1.1166x over previous
import functools

import jax
import jax.numpy as jnp
from jax import lax
from jax.experimental import pallas as pl
from jax.experimental.pallas import tpu as pltpu
from jax.experimental.pallas import tpu_sc as plsc

TILE_T = 512
TILE_M = 512
TILE_F = 512


def _router_body(x_ref, wr_ref, br_ref, gpair_ref, pos1_ref, pos2_ref,
                 gids_ref, cnt_s, base_s, *, n_e, tile_m, n_tiles):
    s = pl.program_id(0)
    t = pl.program_id(1)
    tile_t = x_ref.shape[0]

    @pl.when((s == 0) & (t == 0))
    def _():
        cnt_s[...] = jnp.zeros_like(cnt_s)

    @pl.when((s == 1) & (t == 0))
    def _():
        base_s[...] = jnp.zeros_like(base_s)

    logits = jnp.dot(x_ref[...], wr_ref[...],
                     preferred_element_type=jnp.float32) + br_ref[...]
    col = lax.broadcasted_iota(jnp.int32, logits.shape, 1)
    m1 = jnp.max(logits, axis=1, keepdims=True)
    e1 = jnp.min(jnp.where(logits == m1, col, n_e), axis=1, keepdims=True)
    dropped = jnp.where(col == e1, -1e30, logits)
    m2 = jnp.max(dropped, axis=1, keepdims=True)
    e2 = jnp.min(jnp.where(dropped == m2, col, n_e), axis=1, keepdims=True)

    g1 = 1.0 / (1.0 + jnp.exp(m2 - m1))
    gpair_ref[...] = jnp.concatenate([g1, 1.0 - g1], axis=1)

    sel1 = (col == e1).astype(jnp.float32)
    sel2 = (col == e2).astype(jnp.float32)
    o12 = sel1 + sel2
    tile_cnt = jnp.sum(o12, axis=0, keepdims=True)

    @pl.when(s == 0)
    def _():
        cnt_s[...] += tile_cnt
        pos1_ref[...] = jnp.zeros_like(pos1_ref)
        pos2_ref[...] = jnp.zeros_like(pos2_ref)
        gids_ref[...] = jnp.zeros_like(gids_ref)

    @pl.when(s == 1)
    def _():
        cnt = cnt_s[...]
        cnt_pad = jnp.ceil(cnt * (1.0 / tile_m)) * tile_m
        r = lax.broadcasted_iota(jnp.int32, (n_e, n_e), 0)
        c = lax.broadcasted_iota(jnp.int32, (n_e, n_e), 1)
        tri = (r < c).astype(jnp.float32)
        off = jnp.dot(cnt_pad, tri, preferred_element_type=jnp.float32)
        ends = off + cnt_pad

        tcol = lax.broadcasted_iota(jnp.int32, (gids_ref.shape[0], 1), 0)
        starts = (tcol * tile_m).astype(jnp.float32)
        gid = jnp.sum((ends <= starts).astype(jnp.float32),
                      axis=1, keepdims=True)
        gids_ref[...] = jnp.minimum(gid, n_e - 1).astype(jnp.int32)

        rr = lax.broadcasted_iota(jnp.int32, (tile_t, tile_t), 0)
        cc = lax.broadcasted_iota(jnp.int32, (tile_t, tile_t), 1)
        lstrict = (cc < rr).astype(jnp.float32)
        cum = jnp.dot(lstrict, o12, preferred_element_type=jnp.float32)
        dst = off + base_s[...] + cum
        val1 = jnp.sum(jnp.where(col == e1, dst, 0.0), axis=1, keepdims=True)
        val2 = jnp.sum(jnp.where(col == e2, dst, 0.0), axis=1, keepdims=True)
        pos1_ref[...] = val1.astype(jnp.int32)[None, :, 0:1]
        pos2_ref[...] = val2.astype(jnp.int32)[None, :, 0:1]
        base_s[...] += tile_cnt


def _router(x2d, Wr, br, n_tiles_m):
    T, D = x2d.shape
    E = Wr.shape[1]
    nt = T // TILE_T
    ntm_pad = max(32, n_tiles_m)
    gpair, pos1, pos2, gids = pl.pallas_call(
        functools.partial(_router_body, n_e=E, tile_m=TILE_M,
                          n_tiles=n_tiles_m),
        grid=(2, nt),
        in_specs=[
            pl.BlockSpec((TILE_T, D), lambda s, t: (t, 0)),
            pl.BlockSpec((D, E), lambda s, t: (0, 0)),
            pl.BlockSpec((1, E), lambda s, t: (0, 0)),
        ],
        out_specs=[
            pl.BlockSpec((TILE_T, 2), lambda s, t: (t, 0)),
            pl.BlockSpec((1, TILE_T, 1), lambda s, t: (t, 0, 0)),
            pl.BlockSpec((1, TILE_T, 1), lambda s, t: (t, 0, 0)),
            pl.BlockSpec((ntm_pad, 1), lambda s, t: (0, 0)),
        ],
        out_shape=[
            jax.ShapeDtypeStruct((T, 2), jnp.float32),
            jax.ShapeDtypeStruct((nt, TILE_T, 1), jnp.int32),
            jax.ShapeDtypeStruct((nt, TILE_T, 1), jnp.int32),
            jax.ShapeDtypeStruct((ntm_pad, 1), jnp.int32),
        ],
        scratch_shapes=[
            pltpu.VMEM((1, E), jnp.float32),
            pltpu.VMEM((1, E), jnp.float32),
        ],
        compiler_params=pltpu.CompilerParams(
            dimension_semantics=("arbitrary", "arbitrary"),
        ),
    )(x2d, Wr, br.reshape(1, E))
    return (gpair, pos1.reshape(T), pos2.reshape(T),
            gids.reshape(ntm_pad)[:n_tiles_m])


def _make_dispatch(T, D, xs_rows):
    info = plsc.get_sparse_core_info()
    nw = info.num_cores * info.num_subcores
    tok_w = T // nw
    chunk = 64
    n_chunks = tok_w // chunk
    assert tok_w % chunk == 0
    mesh = plsc.VectorSubcoreMesh(core_axis_name="c", subcore_axis_name="s")

    @functools.partial(
        pl.kernel, mesh=mesh,
        out_type=jax.ShapeDtypeStruct((xs_rows, D), jnp.float32),
        scratch_types=[
            pltpu.VMEM((n_chunks, chunk), jnp.int32),
            pltpu.VMEM((n_chunks, chunk), jnp.int32),
            pltpu.VMEM((chunk, D), jnp.float32),
            pltpu.SemaphoreType.DMA,
        ],
        compiler_params=pltpu.CompilerParams(needs_layout_passes=False),
    )
    def dispatch(x_hbm, pos1_hbm, pos2_hbm, xs_hbm,
                 idx1_v, idx2_v, rows_v, sem):
        wid = lax.axis_index("s") * info.num_cores + lax.axis_index("c")
        pltpu.sync_copy(pos1_hbm.at[pl.ds(wid * n_chunks, n_chunks)], idx1_v)
        pltpu.sync_copy(pos2_hbm.at[pl.ds(wid * n_chunks, n_chunks)], idx2_v)
        for c in range(n_chunks):
            pltpu.sync_copy(
                x_hbm.at[pl.ds(wid * tok_w + c * chunk, chunk)], rows_v)
            cp1 = pltpu.async_copy(rows_v, xs_hbm.at[idx1_v.at[c]], sem)
            cp2 = pltpu.async_copy(rows_v, xs_hbm.at[idx2_v.at[c]], sem)
            cp1.wait()
            cp2.wait()

    return dispatch


def _mlp_body(gids_ref, xs_ref, w1_ref, b1_ref, w2_ref, b2_ref, out_ref):
    f = pl.program_id(1)

    @pl.when(f == 0)
    def _():
        out_ref[...] = jnp.broadcast_to(b2_ref[0], out_ref.shape)

    x16 = xs_ref[...].astype(jnp.bfloat16)
    h = jnp.maximum(
        jnp.dot(x16, w1_ref[0].astype(jnp.bfloat16),
                preferred_element_type=jnp.float32)
        + b1_ref[0], 0.0)
    out_ref[...] += jnp.dot(h.astype(jnp.bfloat16),
                            w2_ref[0].astype(jnp.bfloat16),
                            preferred_element_type=jnp.float32)


def _grouped_mlp(Xs, W1, b1, W2, b2, gids, n_tiles_m):
    E, D, F = W1.shape
    nf = F // TILE_F
    grid_spec = pltpu.PrefetchScalarGridSpec(
        num_scalar_prefetch=1,
        grid=(n_tiles_m, nf),
        in_specs=[
            pl.BlockSpec((TILE_M, D), lambda t, f, g: (t, 0)),
            pl.BlockSpec((1, D, TILE_F), lambda t, f, g: (g[t], 0, f)),
            pl.BlockSpec((1, 1, TILE_F), lambda t, f, g: (g[t], 0, f)),
            pl.BlockSpec((1, TILE_F, D), lambda t, f, g: (g[t], f, 0)),
            pl.BlockSpec((1, 1, D), lambda t, f, g: (g[t], 0, 0)),
        ],
        out_specs=pl.BlockSpec((TILE_M, D), lambda t, f, g: (t, 0)),
    )
    return pl.pallas_call(
        _mlp_body,
        grid_spec=grid_spec,
        out_shape=jax.ShapeDtypeStruct((n_tiles_m * TILE_M, D), jnp.float32),
        compiler_params=pltpu.CompilerParams(
            dimension_semantics=("arbitrary", "arbitrary"),
        ),
    )(gids, Xs, W1, b1.reshape(E, 1, F), W2, b2.reshape(E, 1, D))


def _make_combine_gather(T, D, ys_rows):
    info = plsc.get_sparse_core_info()
    nw = info.num_cores * info.num_subcores
    tok_w = T // nw
    chunk = 64
    n_chunks = tok_w // chunk
    assert tok_w % chunk == 0
    mesh = plsc.VectorSubcoreMesh(core_axis_name="c", subcore_axis_name="s")

    @functools.partial(
        pl.kernel, mesh=mesh,
        out_type=[jax.ShapeDtypeStruct((T, D), jnp.float32),
                  jax.ShapeDtypeStruct((T, D), jnp.float32)],
        scratch_types=[
            pltpu.VMEM((tok_w,), jnp.int32),
            pltpu.VMEM((chunk, D), jnp.float32),
            pltpu.SemaphoreType.DMA,
        ],
        compiler_params=pltpu.CompilerParams(needs_layout_passes=False),
    )
    def combine(ys_hbm, pos1_hbm, pos2_hbm, y1_hbm, y2_hbm,
                idx_v, rows_v, sem):
        wid = lax.axis_index("s") * info.num_cores + lax.axis_index("c")
        base = wid * tok_w
        for pos_hbm, y_hbm in ((pos1_hbm, y1_hbm), (pos2_hbm, y2_hbm)):
            pltpu.sync_copy(pos_hbm.at[pl.ds(base, tok_w)], idx_v)
            for c in range(n_chunks):
                idx = idx_v.at[pl.ds(c * chunk, chunk)]
                pltpu.async_copy(ys_hbm.at[idx], rows_v, sem).wait()
                pltpu.sync_copy(rows_v,
                                y_hbm.at[pl.ds(base + c * chunk, chunk)])

    return combine


def _combine_body(g_ref, y1_ref, y2_ref, out_ref):
    g = g_ref[...]
    out_ref[...] = g[:, 0:1] * y1_ref[...] + g[:, 1:2] * y2_ref[...]


def _combine(gpair, Y1, Y2):
    T, D = Y1.shape
    nt = T // TILE_T
    return pl.pallas_call(
        _combine_body,
        grid=(nt,),
        in_specs=[
            pl.BlockSpec((TILE_T, 2), lambda t: (t, 0)),
            pl.BlockSpec((TILE_T, D), lambda t: (t, 0)),
            pl.BlockSpec((TILE_T, D), lambda t: (t, 0)),
        ],
        out_specs=pl.BlockSpec((TILE_T, D), lambda t: (t, 0)),
        out_shape=jax.ShapeDtypeStruct((T, D), jnp.float32),
    )(gpair, Y1, Y2)


def kernel(x, Wr, br, W1, b1, W2, b2):
    B, S, D = x.shape
    E = Wr.shape[1]
    T = B * S
    x2d = x.reshape(T, D)

    n_tiles_m = 2 * T // TILE_M + E - 1
    xs_rows = n_tiles_m * TILE_M

    gpair, pos1, pos2, gids = _router(x2d, Wr, br, n_tiles_m)
    Xs = _make_dispatch(T, D, xs_rows)(
        x2d, pos1.reshape(T // 64, 64), pos2.reshape(T // 64, 64))
    Ys = _grouped_mlp(Xs, W1, b1, W2, b2, gids, n_tiles_m)
    Y1, Y2 = _make_combine_gather(T, D, Ys.shape[0])(Ys, pos1, pos2)
    out = _combine(gpair, Y1, Y2)
    return out.reshape(B, S, D)


# speedup vs baseline: 1.2619x; 1.1301x over previous
import functools

import jax
import jax.numpy as jnp
from jax import lax
from jax.experimental import pallas as pl
from jax.experimental.pallas import tpu as pltpu
from jax.experimental.pallas import tpu_sc as plsc

TILE_T = 512
TILE_M = 512
TILE_F = 512


def _router_body(x_ref, wr_ref, br_ref, gpair_ref, pos1_ref, pos2_ref,
                 gids_ref, used_ref, cnt_s, base_s, *, n_e, tile_m, n_tiles):
    s = pl.program_id(0)
    t = pl.program_id(1)
    tile_t = x_ref.shape[0]

    @pl.when((s == 0) & (t == 0))
    def _():
        cnt_s[...] = jnp.zeros_like(cnt_s)

    @pl.when((s == 1) & (t == 0))
    def _():
        base_s[...] = jnp.zeros_like(base_s)

    logits = jnp.dot(x_ref[...], wr_ref[...],
                     preferred_element_type=jnp.float32) + br_ref[...]
    col = lax.broadcasted_iota(jnp.int32, logits.shape, 1)
    m1 = jnp.max(logits, axis=1, keepdims=True)
    e1 = jnp.min(jnp.where(logits == m1, col, n_e), axis=1, keepdims=True)
    dropped = jnp.where(col == e1, -1e30, logits)
    m2 = jnp.max(dropped, axis=1, keepdims=True)
    e2 = jnp.min(jnp.where(dropped == m2, col, n_e), axis=1, keepdims=True)

    g1 = 1.0 / (1.0 + jnp.exp(m2 - m1))
    gpair_ref[...] = jnp.concatenate([g1, 1.0 - g1], axis=1)

    sel1 = (col == e1).astype(jnp.float32)
    sel2 = (col == e2).astype(jnp.float32)
    o12 = sel1 + sel2
    tile_cnt = jnp.sum(o12, axis=0, keepdims=True)

    @pl.when(s == 0)
    def _():
        cnt_s[...] += tile_cnt
        pos1_ref[...] = jnp.zeros_like(pos1_ref)
        pos2_ref[...] = jnp.zeros_like(pos2_ref)
        gids_ref[...] = jnp.zeros_like(gids_ref)
        used_ref[...] = jnp.zeros_like(used_ref)

    @pl.when(s == 1)
    def _():
        cnt = cnt_s[...]
        cnt_pad = jnp.ceil(cnt * (1.0 / tile_m)) * tile_m
        r = lax.broadcasted_iota(jnp.int32, (n_e, n_e), 0)
        c = lax.broadcasted_iota(jnp.int32, (n_e, n_e), 1)
        tri = (r < c).astype(jnp.float32)
        off = jnp.dot(cnt_pad, tri, preferred_element_type=jnp.float32)
        ends = off + cnt_pad

        tcol = lax.broadcasted_iota(jnp.int32, (gids_ref.shape[0], 1), 0)
        starts = (tcol * tile_m).astype(jnp.float32)
        gid = jnp.sum((ends <= starts).astype(jnp.float32),
                      axis=1, keepdims=True)
        gids_ref[...] = jnp.minimum(gid, n_e - 1).astype(jnp.int32)
        total_pad = jnp.sum(cnt_pad)
        used_ref[...] = (starts < total_pad).astype(jnp.int32)

        rr = lax.broadcasted_iota(jnp.int32, (tile_t, tile_t), 0)
        cc = lax.broadcasted_iota(jnp.int32, (tile_t, tile_t), 1)
        lstrict = (cc < rr).astype(jnp.float32)
        cum = jnp.dot(lstrict, o12, preferred_element_type=jnp.float32)
        dst = off + base_s[...] + cum
        val1 = jnp.sum(jnp.where(col == e1, dst, 0.0), axis=1, keepdims=True)
        val2 = jnp.sum(jnp.where(col == e2, dst, 0.0), axis=1, keepdims=True)
        pos1_ref[...] = val1.astype(jnp.int32)[None, :, 0:1]
        pos2_ref[...] = val2.astype(jnp.int32)[None, :, 0:1]
        base_s[...] += tile_cnt


def _router(x2d, Wr, br, n_tiles_m):
    T, D = x2d.shape
    E = Wr.shape[1]
    nt = T // TILE_T
    ntm_pad = max(32, n_tiles_m)
    gpair, pos1, pos2, gids, used = pl.pallas_call(
        functools.partial(_router_body, n_e=E, tile_m=TILE_M,
                          n_tiles=n_tiles_m),
        grid=(2, nt),
        in_specs=[
            pl.BlockSpec((TILE_T, D), lambda s, t: (t, 0)),
            pl.BlockSpec((D, E), lambda s, t: (0, 0)),
            pl.BlockSpec((1, E), lambda s, t: (0, 0)),
        ],
        out_specs=[
            pl.BlockSpec((TILE_T, 2), lambda s, t: (t, 0)),
            pl.BlockSpec((1, TILE_T, 1), lambda s, t: (t, 0, 0)),
            pl.BlockSpec((1, TILE_T, 1), lambda s, t: (t, 0, 0)),
            pl.BlockSpec((ntm_pad, 1), lambda s, t: (0, 0)),
            pl.BlockSpec((ntm_pad, 1), lambda s, t: (0, 0)),
        ],
        out_shape=[
            jax.ShapeDtypeStruct((T, 2), jnp.float32),
            jax.ShapeDtypeStruct((nt, TILE_T, 1), jnp.int32),
            jax.ShapeDtypeStruct((nt, TILE_T, 1), jnp.int32),
            jax.ShapeDtypeStruct((ntm_pad, 1), jnp.int32),
            jax.ShapeDtypeStruct((ntm_pad, 1), jnp.int32),
        ],
        scratch_shapes=[
            pltpu.VMEM((1, E), jnp.float32),
            pltpu.VMEM((1, E), jnp.float32),
        ],
        compiler_params=pltpu.CompilerParams(
            dimension_semantics=("arbitrary", "arbitrary"),
        ),
    )(x2d, Wr, br.reshape(1, E))
    return (gpair, pos1.reshape(T), pos2.reshape(T),
            gids.reshape(ntm_pad)[:n_tiles_m],
            used.reshape(ntm_pad)[:n_tiles_m])


def _make_dispatch(T, D, xs_rows):
    info = plsc.get_sparse_core_info()
    nw = info.num_cores * info.num_subcores
    tok_w = T // nw
    chunk = 64
    n_chunks = tok_w // chunk
    assert tok_w % chunk == 0
    mesh = plsc.VectorSubcoreMesh(core_axis_name="c", subcore_axis_name="s")

    @functools.partial(
        pl.kernel, mesh=mesh,
        out_type=jax.ShapeDtypeStruct((xs_rows, D), jnp.float32),
        scratch_types=[
            pltpu.VMEM((n_chunks, chunk), jnp.int32),
            pltpu.VMEM((n_chunks, chunk), jnp.int32),
            pltpu.VMEM((chunk, D), jnp.float32),
            pltpu.SemaphoreType.DMA,
        ],
        compiler_params=pltpu.CompilerParams(needs_layout_passes=False),
    )
    def dispatch(x_hbm, pos1_hbm, pos2_hbm, xs_hbm,
                 idx1_v, idx2_v, rows_v, sem):
        wid = lax.axis_index("s") * info.num_cores + lax.axis_index("c")
        pltpu.sync_copy(pos1_hbm.at[pl.ds(wid * n_chunks, n_chunks)], idx1_v)
        pltpu.sync_copy(pos2_hbm.at[pl.ds(wid * n_chunks, n_chunks)], idx2_v)
        for c in range(n_chunks):
            pltpu.sync_copy(
                x_hbm.at[pl.ds(wid * tok_w + c * chunk, chunk)], rows_v)
            cp1 = pltpu.async_copy(rows_v, xs_hbm.at[idx1_v.at[c]], sem)
            cp2 = pltpu.async_copy(rows_v, xs_hbm.at[idx2_v.at[c]], sem)
            cp1.wait()
            cp2.wait()

    return dispatch


def _mlp_body(gids_ref, used_ref, xs_ref, w1_ref, b1_ref, w2_ref, b2_ref,
              out_ref):
    t = pl.program_id(0)
    f = pl.program_id(1)

    @pl.when(used_ref[t] != 0)
    def _():
        @pl.when(f == 0)
        def _():
            out_ref[...] = jnp.broadcast_to(b2_ref[0], out_ref.shape)

        h = jnp.maximum(
            jnp.dot(xs_ref[...], w1_ref[0],
                    preferred_element_type=jnp.float32) + b1_ref[0], 0.0)
        out_ref[...] += jnp.dot(h, w2_ref[0],
                                preferred_element_type=jnp.float32)


def _grouped_mlp(Xs, W1, b1, W2, b2, gids, used, n_tiles_m):
    E, D, F = W1.shape
    nf = F // TILE_F
    grid_spec = pltpu.PrefetchScalarGridSpec(
        num_scalar_prefetch=2,
        grid=(n_tiles_m, nf),
        in_specs=[
            pl.BlockSpec((TILE_M, D), lambda t, f, g, u: (t * u[t], 0)),
            pl.BlockSpec((1, D, TILE_F), lambda t, f, g, u: (g[t], 0, f * u[t])),
            pl.BlockSpec((1, 1, TILE_F), lambda t, f, g, u: (g[t], 0, f * u[t])),
            pl.BlockSpec((1, TILE_F, D), lambda t, f, g, u: (g[t], f * u[t], 0)),
            pl.BlockSpec((1, 1, D), lambda t, f, g, u: (g[t], 0, 0)),
        ],
        out_specs=pl.BlockSpec((TILE_M, D), lambda t, f, g, u: (t, 0)),
    )
    return pl.pallas_call(
        _mlp_body,
        grid_spec=grid_spec,
        out_shape=jax.ShapeDtypeStruct((n_tiles_m * TILE_M, D), jnp.float32),
        compiler_params=pltpu.CompilerParams(
            dimension_semantics=("arbitrary", "arbitrary"),
        ),
    )(gids, used, Xs, W1, b1.reshape(E, 1, F), W2, b2.reshape(E, 1, D))


def _make_combine_gather(T, D, ys_rows):
    info = plsc.get_sparse_core_info()
    nw = info.num_cores * info.num_subcores
    tok_w = T // nw
    chunk = 64
    n_chunks = tok_w // chunk
    assert tok_w % chunk == 0
    mesh = plsc.VectorSubcoreMesh(core_axis_name="c", subcore_axis_name="s")

    @functools.partial(
        pl.kernel, mesh=mesh,
        out_type=[jax.ShapeDtypeStruct((T, D), jnp.float32),
                  jax.ShapeDtypeStruct((T, D), jnp.float32)],
        scratch_types=[
            pltpu.VMEM((tok_w,), jnp.int32),
            pltpu.VMEM((chunk, D), jnp.float32),
            pltpu.SemaphoreType.DMA,
        ],
        compiler_params=pltpu.CompilerParams(needs_layout_passes=False),
    )
    def combine(ys_hbm, pos1_hbm, pos2_hbm, y1_hbm, y2_hbm,
                idx_v, rows_v, sem):
        wid = lax.axis_index("s") * info.num_cores + lax.axis_index("c")
        base = wid * tok_w
        for pos_hbm, y_hbm in ((pos1_hbm, y1_hbm), (pos2_hbm, y2_hbm)):
            pltpu.sync_copy(pos_hbm.at[pl.ds(base, tok_w)], idx_v)
            for c in range(n_chunks):
                idx = idx_v.at[pl.ds(c * chunk, chunk)]
                pltpu.async_copy(ys_hbm.at[idx], rows_v, sem).wait()
                pltpu.sync_copy(rows_v,
                                y_hbm.at[pl.ds(base + c * chunk, chunk)])

    return combine


def _combine_body(g_ref, y1_ref, y2_ref, out_ref):
    g = g_ref[...]
    out_ref[...] = g[:, 0:1] * y1_ref[...] + g[:, 1:2] * y2_ref[...]


def _combine(gpair, Y1, Y2):
    T, D = Y1.shape
    nt = T // TILE_T
    return pl.pallas_call(
        _combine_body,
        grid=(nt,),
        in_specs=[
            pl.BlockSpec((TILE_T, 2), lambda t: (t, 0)),
            pl.BlockSpec((TILE_T, D), lambda t: (t, 0)),
            pl.BlockSpec((TILE_T, D), lambda t: (t, 0)),
        ],
        out_specs=pl.BlockSpec((TILE_T, D), lambda t: (t, 0)),
        out_shape=jax.ShapeDtypeStruct((T, D), jnp.float32),
    )(gpair, Y1, Y2)


def kernel(x, Wr, br, W1, b1, W2, b2):
    B, S, D = x.shape
    E = Wr.shape[1]
    T = B * S
    x2d = x.reshape(T, D)

    n_tiles_m = 2 * T // TILE_M + E - 1
    xs_rows = n_tiles_m * TILE_M

    gpair, pos1, pos2, gids, used = _router(x2d, Wr, br, n_tiles_m)
    Xs = _make_dispatch(T, D, xs_rows)(
        x2d, pos1.reshape(T // 64, 64), pos2.reshape(T // 64, 64))
    Ys = _grouped_mlp(Xs, W1, b1, W2, b2, gids, used, n_tiles_m)
    Y1, Y2 = _make_combine_gather(T, D, Ys.shape[0])(Ys, pos1, pos2)
    out = _combine(gpair, Y1, Y2)
    return out.reshape(B, S, D)


# speedup vs baseline: 1.4174x; 1.1232x over previous
import functools

import jax
import jax.numpy as jnp
from jax import lax
from jax.experimental import pallas as pl
from jax.experimental.pallas import tpu as pltpu
from jax.experimental.pallas import tpu_sc as plsc

TILE_T = 512
TILE_M = 512
TILE_F = 1024


def _router_body(x_ref, wr_ref, br_ref, gpair_ref, pos1_ref, pos2_ref,
                 gids_ref, used_ref, cnt_s, base_s, *, n_e, tile_m, n_tiles):
    s = pl.program_id(0)
    t = pl.program_id(1)
    tile_t = x_ref.shape[0]

    @pl.when((s == 0) & (t == 0))
    def _():
        cnt_s[...] = jnp.zeros_like(cnt_s)

    @pl.when((s == 1) & (t == 0))
    def _():
        base_s[...] = jnp.zeros_like(base_s)

    logits = jnp.dot(x_ref[...], wr_ref[...],
                     preferred_element_type=jnp.float32) + br_ref[...]
    col = lax.broadcasted_iota(jnp.int32, logits.shape, 1)
    m1 = jnp.max(logits, axis=1, keepdims=True)
    e1 = jnp.min(jnp.where(logits == m1, col, n_e), axis=1, keepdims=True)
    dropped = jnp.where(col == e1, -1e30, logits)
    m2 = jnp.max(dropped, axis=1, keepdims=True)
    e2 = jnp.min(jnp.where(dropped == m2, col, n_e), axis=1, keepdims=True)

    g1 = 1.0 / (1.0 + jnp.exp(m2 - m1))
    gpair_ref[...] = jnp.concatenate([g1, 1.0 - g1], axis=1)

    sel1 = (col == e1).astype(jnp.float32)
    sel2 = (col == e2).astype(jnp.float32)
    o12 = sel1 + sel2
    tile_cnt = jnp.sum(o12, axis=0, keepdims=True)

    @pl.when(s == 0)
    def _():
        cnt_s[...] += tile_cnt
        pos1_ref[...] = jnp.zeros_like(pos1_ref)
        pos2_ref[...] = jnp.zeros_like(pos2_ref)
        gids_ref[...] = jnp.zeros_like(gids_ref)
        used_ref[...] = jnp.zeros_like(used_ref)

    @pl.when(s == 1)
    def _():
        cnt = cnt_s[...]
        cnt_pad = jnp.ceil(cnt * (1.0 / tile_m)) * tile_m
        r = lax.broadcasted_iota(jnp.int32, (n_e, n_e), 0)
        c = lax.broadcasted_iota(jnp.int32, (n_e, n_e), 1)
        tri = (r < c).astype(jnp.float32)
        off = jnp.dot(cnt_pad, tri, preferred_element_type=jnp.float32)
        ends = off + cnt_pad

        tcol = lax.broadcasted_iota(jnp.int32, (gids_ref.shape[0], 1), 0)
        starts = (tcol * tile_m).astype(jnp.float32)
        gid = jnp.sum((ends <= starts).astype(jnp.float32),
                      axis=1, keepdims=True)
        gids_ref[...] = jnp.minimum(gid, n_e - 1).astype(jnp.int32)
        total_pad = jnp.sum(cnt_pad)
        used_ref[...] = (starts < total_pad).astype(jnp.int32)

        rr = lax.broadcasted_iota(jnp.int32, (tile_t, tile_t), 0)
        cc = lax.broadcasted_iota(jnp.int32, (tile_t, tile_t), 1)
        lstrict = (cc < rr).astype(jnp.float32)
        cum = jnp.dot(lstrict, o12, preferred_element_type=jnp.float32)
        dst = off + base_s[...] + cum
        val1 = jnp.sum(jnp.where(col == e1, dst, 0.0), axis=1, keepdims=True)
        val2 = jnp.sum(jnp.where(col == e2, dst, 0.0), axis=1, keepdims=True)
        pos1_ref[...] = val1.astype(jnp.int32)[None, :, 0:1]
        pos2_ref[...] = val2.astype(jnp.int32)[None, :, 0:1]
        base_s[...] += tile_cnt


def _router(x2d, Wr, br, n_tiles_m):
    T, D = x2d.shape
    E = Wr.shape[1]
    nt = T // TILE_T
    ntm_pad = max(32, n_tiles_m)
    gpair, pos1, pos2, gids, used = pl.pallas_call(
        functools.partial(_router_body, n_e=E, tile_m=TILE_M,
                          n_tiles=n_tiles_m),
        grid=(2, nt),
        in_specs=[
            pl.BlockSpec((TILE_T, D), lambda s, t: (t, 0)),
            pl.BlockSpec((D, E), lambda s, t: (0, 0)),
            pl.BlockSpec((1, E), lambda s, t: (0, 0)),
        ],
        out_specs=[
            pl.BlockSpec((TILE_T, 2), lambda s, t: (t, 0)),
            pl.BlockSpec((1, TILE_T, 1), lambda s, t: (t, 0, 0)),
            pl.BlockSpec((1, TILE_T, 1), lambda s, t: (t, 0, 0)),
            pl.BlockSpec((ntm_pad, 1), lambda s, t: (0, 0)),
            pl.BlockSpec((ntm_pad, 1), lambda s, t: (0, 0)),
        ],
        out_shape=[
            jax.ShapeDtypeStruct((T, 2), jnp.float32),
            jax.ShapeDtypeStruct((nt, TILE_T, 1), jnp.int32),
            jax.ShapeDtypeStruct((nt, TILE_T, 1), jnp.int32),
            jax.ShapeDtypeStruct((ntm_pad, 1), jnp.int32),
            jax.ShapeDtypeStruct((ntm_pad, 1), jnp.int32),
        ],
        scratch_shapes=[
            pltpu.VMEM((1, E), jnp.float32),
            pltpu.VMEM((1, E), jnp.float32),
        ],
        compiler_params=pltpu.CompilerParams(
            dimension_semantics=("arbitrary", "arbitrary"),
        ),
    )(x2d, Wr, br.reshape(1, E))
    return (gpair, pos1.reshape(T), pos2.reshape(T),
            gids.reshape(ntm_pad)[:n_tiles_m],
            used.reshape(ntm_pad)[:n_tiles_m])


def _make_dispatch(T, D, xs_rows):
    info = plsc.get_sparse_core_info()
    nw = info.num_cores * info.num_subcores
    tok_w = T // nw
    chunk = 64
    n_chunks = tok_w // chunk
    assert tok_w % chunk == 0
    mesh = plsc.VectorSubcoreMesh(core_axis_name="c", subcore_axis_name="s")

    @functools.partial(
        pl.kernel, mesh=mesh,
        out_type=jax.ShapeDtypeStruct((xs_rows, D), jnp.float32),
        scratch_types=[
            pltpu.VMEM((n_chunks, chunk), jnp.int32),
            pltpu.VMEM((n_chunks, chunk), jnp.int32),
            pltpu.VMEM((chunk, D), jnp.float32),
            pltpu.SemaphoreType.DMA,
        ],
        compiler_params=pltpu.CompilerParams(needs_layout_passes=False),
    )
    def dispatch(x_hbm, pos1_hbm, pos2_hbm, xs_hbm,
                 idx1_v, idx2_v, rows_v, sem):
        wid = lax.axis_index("s") * info.num_cores + lax.axis_index("c")
        pltpu.sync_copy(pos1_hbm.at[pl.ds(wid * n_chunks, n_chunks)], idx1_v)
        pltpu.sync_copy(pos2_hbm.at[pl.ds(wid * n_chunks, n_chunks)], idx2_v)
        for c in range(n_chunks):
            pltpu.sync_copy(
                x_hbm.at[pl.ds(wid * tok_w + c * chunk, chunk)], rows_v)
            cp1 = pltpu.async_copy(rows_v, xs_hbm.at[idx1_v.at[c]], sem)
            cp2 = pltpu.async_copy(rows_v, xs_hbm.at[idx2_v.at[c]], sem)
            cp1.wait()
            cp2.wait()

    return dispatch


def _mlp_body(gids_ref, used_ref, xs_ref, w1_ref, b1_ref, w2_ref, b2_ref,
              out_ref):
    t = pl.program_id(0)
    f = pl.program_id(1)

    @pl.when(used_ref[t] != 0)
    def _():
        @pl.when(f == 0)
        def _():
            out_ref[...] = jnp.broadcast_to(b2_ref[0], out_ref.shape)

        h = jnp.maximum(
            jnp.dot(xs_ref[...], w1_ref[0],
                    preferred_element_type=jnp.float32) + b1_ref[0], 0.0)
        out_ref[...] += jnp.dot(h, w2_ref[0],
                                preferred_element_type=jnp.float32)


def _grouped_mlp(Xs, W1, b1, W2, b2, gids, used, n_tiles_m):
    E, D, F = W1.shape
    nf = F // TILE_F
    grid_spec = pltpu.PrefetchScalarGridSpec(
        num_scalar_prefetch=2,
        grid=(n_tiles_m, nf),
        in_specs=[
            pl.BlockSpec((TILE_M, D), lambda t, f, g, u: (t * u[t], 0)),
            pl.BlockSpec((1, D, TILE_F), lambda t, f, g, u: (g[t], 0, f * u[t])),
            pl.BlockSpec((1, 1, TILE_F), lambda t, f, g, u: (g[t], 0, f * u[t])),
            pl.BlockSpec((1, TILE_F, D), lambda t, f, g, u: (g[t], f * u[t], 0)),
            pl.BlockSpec((1, 1, D), lambda t, f, g, u: (g[t], 0, 0)),
        ],
        out_specs=pl.BlockSpec((TILE_M, D), lambda t, f, g, u: (t, 0)),
    )
    return pl.pallas_call(
        _mlp_body,
        grid_spec=grid_spec,
        out_shape=jax.ShapeDtypeStruct((n_tiles_m * TILE_M, D), jnp.float32),
        compiler_params=pltpu.CompilerParams(
            dimension_semantics=("arbitrary", "arbitrary"),
        ),
    )(gids, used, Xs, W1, b1.reshape(E, 1, F), W2, b2.reshape(E, 1, D))


def _make_combine_gather(T, D, ys_rows):
    info = plsc.get_sparse_core_info()
    nw = info.num_cores * info.num_subcores
    tok_w = T // nw
    chunk = 64
    n_chunks = tok_w // chunk
    assert tok_w % chunk == 0
    mesh = plsc.VectorSubcoreMesh(core_axis_name="c", subcore_axis_name="s")

    @functools.partial(
        pl.kernel, mesh=mesh,
        out_type=[jax.ShapeDtypeStruct((T, D), jnp.float32),
                  jax.ShapeDtypeStruct((T, D), jnp.float32)],
        scratch_types=[
            pltpu.VMEM((tok_w,), jnp.int32),
            pltpu.VMEM((chunk, D), jnp.float32),
            pltpu.SemaphoreType.DMA,
        ],
        compiler_params=pltpu.CompilerParams(needs_layout_passes=False),
    )
    def combine(ys_hbm, pos1_hbm, pos2_hbm, y1_hbm, y2_hbm,
                idx_v, rows_v, sem):
        wid = lax.axis_index("s") * info.num_cores + lax.axis_index("c")
        base = wid * tok_w
        for pos_hbm, y_hbm in ((pos1_hbm, y1_hbm), (pos2_hbm, y2_hbm)):
            pltpu.sync_copy(pos_hbm.at[pl.ds(base, tok_w)], idx_v)
            for c in range(n_chunks):
                idx = idx_v.at[pl.ds(c * chunk, chunk)]
                pltpu.async_copy(ys_hbm.at[idx], rows_v, sem).wait()
                pltpu.sync_copy(rows_v,
                                y_hbm.at[pl.ds(base + c * chunk, chunk)])

    return combine


def _combine_body(g_ref, y1_ref, y2_ref, out_ref):
    g = g_ref[...]
    out_ref[...] = g[:, 0:1] * y1_ref[...] + g[:, 1:2] * y2_ref[...]


def _combine(gpair, Y1, Y2):
    T, D = Y1.shape
    nt = T // TILE_T
    return pl.pallas_call(
        _combine_body,
        grid=(nt,),
        in_specs=[
            pl.BlockSpec((TILE_T, 2), lambda t: (t, 0)),
            pl.BlockSpec((TILE_T, D), lambda t: (t, 0)),
            pl.BlockSpec((TILE_T, D), lambda t: (t, 0)),
        ],
        out_specs=pl.BlockSpec((TILE_T, D), lambda t: (t, 0)),
        out_shape=jax.ShapeDtypeStruct((T, D), jnp.float32),
    )(gpair, Y1, Y2)


def kernel(x, Wr, br, W1, b1, W2, b2):
    B, S, D = x.shape
    E = Wr.shape[1]
    T = B * S
    x2d = x.reshape(T, D)

    n_tiles_m = 2 * T // TILE_M + E - 1
    xs_rows = n_tiles_m * TILE_M

    gpair, pos1, pos2, gids, used = _router(x2d, Wr, br, n_tiles_m)
    Xs = _make_dispatch(T, D, xs_rows)(
        x2d, pos1.reshape(T // 64, 64), pos2.reshape(T // 64, 64))
    Ys = _grouped_mlp(Xs, W1, b1, W2, b2, gids, used, n_tiles_m)
    Y1, Y2 = _make_combine_gather(T, D, Ys.shape[0])(Ys, pos1, pos2)
    out = _combine(gpair, Y1, Y2)
    return out.reshape(B, S, D)


# speedup vs baseline: 1.4818x; 1.0455x over previous
import functools

import jax
import jax.numpy as jnp
from jax import lax
from jax.experimental import pallas as pl
from jax.experimental.pallas import tpu as pltpu
from jax.experimental.pallas import tpu_sc as plsc

TILE_T = 512
TILE_M = 512
TILE_F = 2048


def _router_body(x_ref, wr_ref, br_ref, gpair_ref, pos1_ref, pos2_ref,
                 gids_ref, used_ref, cnt_s, base_s, *, n_e, tile_m, n_tiles):
    s = pl.program_id(0)
    t = pl.program_id(1)
    tile_t = x_ref.shape[0]

    @pl.when((s == 0) & (t == 0))
    def _():
        cnt_s[...] = jnp.zeros_like(cnt_s)

    @pl.when((s == 1) & (t == 0))
    def _():
        base_s[...] = jnp.zeros_like(base_s)

    logits = jnp.dot(x_ref[...], wr_ref[...],
                     preferred_element_type=jnp.float32) + br_ref[...]
    col = lax.broadcasted_iota(jnp.int32, logits.shape, 1)
    m1 = jnp.max(logits, axis=1, keepdims=True)
    e1 = jnp.min(jnp.where(logits == m1, col, n_e), axis=1, keepdims=True)
    dropped = jnp.where(col == e1, -1e30, logits)
    m2 = jnp.max(dropped, axis=1, keepdims=True)
    e2 = jnp.min(jnp.where(dropped == m2, col, n_e), axis=1, keepdims=True)

    g1 = 1.0 / (1.0 + jnp.exp(m2 - m1))
    gpair_ref[...] = jnp.concatenate([g1, 1.0 - g1], axis=1)

    sel1 = (col == e1).astype(jnp.float32)
    sel2 = (col == e2).astype(jnp.float32)
    o12 = sel1 + sel2
    tile_cnt = jnp.sum(o12, axis=0, keepdims=True)

    @pl.when(s == 0)
    def _():
        cnt_s[...] += tile_cnt
        pos1_ref[...] = jnp.zeros_like(pos1_ref)
        pos2_ref[...] = jnp.zeros_like(pos2_ref)
        gids_ref[...] = jnp.zeros_like(gids_ref)
        used_ref[...] = jnp.zeros_like(used_ref)

    @pl.when(s == 1)
    def _():
        cnt = cnt_s[...]
        cnt_pad = jnp.ceil(cnt * (1.0 / tile_m)) * tile_m
        r = lax.broadcasted_iota(jnp.int32, (n_e, n_e), 0)
        c = lax.broadcasted_iota(jnp.int32, (n_e, n_e), 1)
        tri = (r < c).astype(jnp.float32)
        off = jnp.dot(cnt_pad, tri, preferred_element_type=jnp.float32)
        ends = off + cnt_pad

        tcol = lax.broadcasted_iota(jnp.int32, (gids_ref.shape[0], 1), 0)
        starts = (tcol * tile_m).astype(jnp.float32)
        gid = jnp.sum((ends <= starts).astype(jnp.float32),
                      axis=1, keepdims=True)
        gids_ref[...] = jnp.minimum(gid, n_e - 1).astype(jnp.int32)
        total_pad = jnp.sum(cnt_pad)
        used_ref[...] = (starts < total_pad).astype(jnp.int32)

        rr = lax.broadcasted_iota(jnp.int32, (tile_t, tile_t), 0)
        cc = lax.broadcasted_iota(jnp.int32, (tile_t, tile_t), 1)
        lstrict = (cc < rr).astype(jnp.float32)
        cum = jnp.dot(lstrict, o12, preferred_element_type=jnp.float32)
        dst = off + base_s[...] + cum
        val1 = jnp.sum(jnp.where(col == e1, dst, 0.0), axis=1, keepdims=True)
        val2 = jnp.sum(jnp.where(col == e2, dst, 0.0), axis=1, keepdims=True)
        pos1_ref[...] = val1.astype(jnp.int32)[None, :, 0:1]
        pos2_ref[...] = val2.astype(jnp.int32)[None, :, 0:1]
        base_s[...] += tile_cnt


def _router(x2d, Wr, br, n_tiles_m):
    T, D = x2d.shape
    E = Wr.shape[1]
    nt = T // TILE_T
    ntm_pad = max(32, n_tiles_m)
    gpair, pos1, pos2, gids, used = pl.pallas_call(
        functools.partial(_router_body, n_e=E, tile_m=TILE_M,
                          n_tiles=n_tiles_m),
        grid=(2, nt),
        in_specs=[
            pl.BlockSpec((TILE_T, D), lambda s, t: (t, 0)),
            pl.BlockSpec((D, E), lambda s, t: (0, 0)),
            pl.BlockSpec((1, E), lambda s, t: (0, 0)),
        ],
        out_specs=[
            pl.BlockSpec((TILE_T, 2), lambda s, t: (t, 0)),
            pl.BlockSpec((1, TILE_T, 1), lambda s, t: (t, 0, 0)),
            pl.BlockSpec((1, TILE_T, 1), lambda s, t: (t, 0, 0)),
            pl.BlockSpec((ntm_pad, 1), lambda s, t: (0, 0)),
            pl.BlockSpec((ntm_pad, 1), lambda s, t: (0, 0)),
        ],
        out_shape=[
            jax.ShapeDtypeStruct((T, 2), jnp.float32),
            jax.ShapeDtypeStruct((nt, TILE_T, 1), jnp.int32),
            jax.ShapeDtypeStruct((nt, TILE_T, 1), jnp.int32),
            jax.ShapeDtypeStruct((ntm_pad, 1), jnp.int32),
            jax.ShapeDtypeStruct((ntm_pad, 1), jnp.int32),
        ],
        scratch_shapes=[
            pltpu.VMEM((1, E), jnp.float32),
            pltpu.VMEM((1, E), jnp.float32),
        ],
        compiler_params=pltpu.CompilerParams(
            dimension_semantics=("arbitrary", "arbitrary"),
        ),
    )(x2d, Wr, br.reshape(1, E))
    return (gpair, pos1.reshape(T), pos2.reshape(T),
            gids.reshape(ntm_pad)[:n_tiles_m],
            used.reshape(ntm_pad)[:n_tiles_m])


def _make_dispatch(T, D, xs_rows):
    info = plsc.get_sparse_core_info()
    nw = info.num_cores * info.num_subcores
    tok_w = T // nw
    chunk = 64
    n_chunks = tok_w // chunk
    assert tok_w % chunk == 0
    mesh = plsc.VectorSubcoreMesh(core_axis_name="c", subcore_axis_name="s")

    @functools.partial(
        pl.kernel, mesh=mesh,
        out_type=jax.ShapeDtypeStruct((xs_rows, D), jnp.float32),
        scratch_types=[
            pltpu.VMEM((n_chunks, chunk), jnp.int32),
            pltpu.VMEM((n_chunks, chunk), jnp.int32),
            pltpu.VMEM((chunk, D), jnp.float32),
            pltpu.SemaphoreType.DMA,
        ],
        compiler_params=pltpu.CompilerParams(needs_layout_passes=False),
    )
    def dispatch(x_hbm, pos1_hbm, pos2_hbm, xs_hbm,
                 idx1_v, idx2_v, rows_v, sem):
        wid = lax.axis_index("s") * info.num_cores + lax.axis_index("c")
        pltpu.sync_copy(pos1_hbm.at[pl.ds(wid * n_chunks, n_chunks)], idx1_v)
        pltpu.sync_copy(pos2_hbm.at[pl.ds(wid * n_chunks, n_chunks)], idx2_v)
        for c in range(n_chunks):
            pltpu.sync_copy(
                x_hbm.at[pl.ds(wid * tok_w + c * chunk, chunk)], rows_v)
            cp1 = pltpu.async_copy(rows_v, xs_hbm.at[idx1_v.at[c]], sem)
            cp2 = pltpu.async_copy(rows_v, xs_hbm.at[idx2_v.at[c]], sem)
            cp1.wait()
            cp2.wait()

    return dispatch


def _mlp_body(gids_ref, used_ref, xs_ref, w1_ref, b1_ref, w2_ref, b2_ref,
              out_ref):
    t = pl.program_id(0)
    f = pl.program_id(1)

    @pl.when(used_ref[t] != 0)
    def _():
        @pl.when(f == 0)
        def _():
            out_ref[...] = jnp.broadcast_to(b2_ref[0], out_ref.shape)

        h = jnp.maximum(
            jnp.dot(xs_ref[...], w1_ref[0],
                    preferred_element_type=jnp.float32) + b1_ref[0], 0.0)
        out_ref[...] += jnp.dot(h, w2_ref[0],
                                preferred_element_type=jnp.float32)


def _grouped_mlp(Xs, W1, b1, W2, b2, gids, used, n_tiles_m):
    E, D, F = W1.shape
    nf = F // TILE_F
    grid_spec = pltpu.PrefetchScalarGridSpec(
        num_scalar_prefetch=2,
        grid=(n_tiles_m, nf),
        in_specs=[
            pl.BlockSpec((TILE_M, D), lambda t, f, g, u: (t * u[t], 0)),
            pl.BlockSpec((1, D, TILE_F), lambda t, f, g, u: (g[t], 0, f * u[t])),
            pl.BlockSpec((1, 1, TILE_F), lambda t, f, g, u: (g[t], 0, f * u[t])),
            pl.BlockSpec((1, TILE_F, D), lambda t, f, g, u: (g[t], f * u[t], 0)),
            pl.BlockSpec((1, 1, D), lambda t, f, g, u: (g[t], 0, 0)),
        ],
        out_specs=pl.BlockSpec((TILE_M, D), lambda t, f, g, u: (t, 0)),
    )
    return pl.pallas_call(
        _mlp_body,
        grid_spec=grid_spec,
        out_shape=jax.ShapeDtypeStruct((n_tiles_m * TILE_M, D), jnp.float32),
        compiler_params=pltpu.CompilerParams(
            dimension_semantics=("arbitrary", "arbitrary"),
        ),
    )(gids, used, Xs, W1, b1.reshape(E, 1, F), W2, b2.reshape(E, 1, D))


def _make_combine_gather(T, D, ys_rows):
    info = plsc.get_sparse_core_info()
    nw = info.num_cores * info.num_subcores
    tok_w = T // nw
    chunk = 64
    n_chunks = tok_w // chunk
    assert tok_w % chunk == 0
    mesh = plsc.VectorSubcoreMesh(core_axis_name="c", subcore_axis_name="s")

    @functools.partial(
        pl.kernel, mesh=mesh,
        out_type=[jax.ShapeDtypeStruct((T, D), jnp.float32),
                  jax.ShapeDtypeStruct((T, D), jnp.float32)],
        scratch_types=[
            pltpu.VMEM((tok_w,), jnp.int32),
            pltpu.VMEM((chunk, D), jnp.float32),
            pltpu.SemaphoreType.DMA,
        ],
        compiler_params=pltpu.CompilerParams(needs_layout_passes=False),
    )
    def combine(ys_hbm, pos1_hbm, pos2_hbm, y1_hbm, y2_hbm,
                idx_v, rows_v, sem):
        wid = lax.axis_index("s") * info.num_cores + lax.axis_index("c")
        base = wid * tok_w
        for pos_hbm, y_hbm in ((pos1_hbm, y1_hbm), (pos2_hbm, y2_hbm)):
            pltpu.sync_copy(pos_hbm.at[pl.ds(base, tok_w)], idx_v)
            for c in range(n_chunks):
                idx = idx_v.at[pl.ds(c * chunk, chunk)]
                pltpu.async_copy(ys_hbm.at[idx], rows_v, sem).wait()
                pltpu.sync_copy(rows_v,
                                y_hbm.at[pl.ds(base + c * chunk, chunk)])

    return combine


def _combine_body(g_ref, y1_ref, y2_ref, out_ref):
    g = g_ref[...]
    out_ref[...] = g[:, 0:1] * y1_ref[...] + g[:, 1:2] * y2_ref[...]


def _combine(gpair, Y1, Y2):
    T, D = Y1.shape
    nt = T // TILE_T
    return pl.pallas_call(
        _combine_body,
        grid=(nt,),
        in_specs=[
            pl.BlockSpec((TILE_T, 2), lambda t: (t, 0)),
            pl.BlockSpec((TILE_T, D), lambda t: (t, 0)),
            pl.BlockSpec((TILE_T, D), lambda t: (t, 0)),
        ],
        out_specs=pl.BlockSpec((TILE_T, D), lambda t: (t, 0)),
        out_shape=jax.ShapeDtypeStruct((T, D), jnp.float32),
    )(gpair, Y1, Y2)


def kernel(x, Wr, br, W1, b1, W2, b2):
    B, S, D = x.shape
    E = Wr.shape[1]
    T = B * S
    x2d = x.reshape(T, D)

    n_tiles_m = 2 * T // TILE_M + E - 1
    xs_rows = n_tiles_m * TILE_M

    gpair, pos1, pos2, gids, used = _router(x2d, Wr, br, n_tiles_m)
    Xs = _make_dispatch(T, D, xs_rows)(
        x2d, pos1.reshape(T // 64, 64), pos2.reshape(T // 64, 64))
    Ys = _grouped_mlp(Xs, W1, b1, W2, b2, gids, used, n_tiles_m)
    Y1, Y2 = _make_combine_gather(T, D, Ys.shape[0])(Ys, pos1, pos2)
    out = _combine(gpair, Y1, Y2)
    return out.reshape(B, S, D)
